# Initial kernel scaffold; baseline (speedup 1.0000x reference)
import jax
import jax.numpy as jnp
from jax.experimental import pallas as pl


def kernel(x, grad_map):
    raise NotImplementedError("write your pallas kernel here")


# speedup vs baseline: 6.0492x; 6.0492x over previous
import math

import jax
import jax.numpy as jnp
import numpy as np
from jax.experimental import pallas as pl
from jax.experimental.pallas import tpu as pltpu

_NUM_CLUSTERS = 196
_H = 384
_W = 384
_S = 27
_MS = 10.0 / 27
_GRID = 14


def _grid_coords():
    num_cols = int(math.sqrt(_NUM_CLUSTERS * _W / _H))
    num_rows = int(math.ceil(_NUM_CLUSTERS / num_cols))
    gy = _H / num_rows
    gx = _W / num_cols
    ys = [int((i + 0.5) * gy) for i in range(num_rows)]
    xs = [int((j + 0.5) * gx) for j in range(num_cols)]
    return ys, xs


_YS, _XS = _grid_coords()


def _assign_body(yc_ref, xc_ref, col_ref, x_ref, lm_ref, dm_ref):
    dm_ref[...] = jnp.full((_H, _W), jnp.inf, jnp.float32)
    lm_ref[...] = jnp.full((1, _H, _W), -1, jnp.int32)

    def step(c, carry):
        cy = yc_ref[0, 0, c]
        cx = xc_ref[0, 0, c]
        c0 = col_ref[0, 0, c]
        c1 = col_ref[0, 1, c]
        c2 = col_ref[0, 2, c]
        r8 = pl.multiple_of(jnp.clip(((cy - _S) // 8) * 8, 0, _H - 64), 8)
        cl = pl.multiple_of(jnp.clip(((cx - _S) // 128) * 128, 0, _W - 256), 128)
        rows = r8 + jax.lax.broadcasted_iota(jnp.int32, (64, 256), 0)
        cols = cl + jax.lax.broadcasted_iota(jnp.int32, (64, 256), 1)
        d0 = x_ref[0, 0, pl.ds(r8, 64), pl.ds(cl, 256)] - c0
        d1 = x_ref[0, 1, pl.ds(r8, 64), pl.ds(cl, 256)] - c1
        d2 = x_ref[0, 2, pl.ds(r8, 64), pl.ds(cl, 256)] - c2
        cd = jnp.sqrt(d0 * d0 + d1 * d1 + d2 * d2)
        dr = rows - cy
        dc = cols - cx
        sp = jnp.sqrt((dr * dr + dc * dc).astype(jnp.float32))
        inw = (
            (rows >= cy - _S) & (rows < cy + _S)
            & (cols >= cx - _S) & (cols < cx + _S)
        )
        dist = jnp.where(inw, cd + jnp.float32(_MS) * sp, jnp.inf)
        dmw = dm_ref[pl.ds(r8, 64), pl.ds(cl, 256)]
        lmw = lm_ref[0, pl.ds(r8, 64), pl.ds(cl, 256)]
        mr = dmw > dist
        dm_ref[pl.ds(r8, 64), pl.ds(cl, 256)] = jnp.where(mr, dist, dmw)
        lm_ref[0, pl.ds(r8, 64), pl.ds(cl, 256)] = jnp.where(mr, c, lmw)
        return carry

    jax.lax.fori_loop(0, _NUM_CLUSTERS, step, 0)


def _assign(x, yc, xc, colors):
    batch = x.shape[0]
    colors_t = jnp.transpose(colors, (0, 2, 1))
    return pl.pallas_call(
        _assign_body,
        grid=(batch,),
        in_specs=[
            pl.BlockSpec((1, 1, _NUM_CLUSTERS), lambda b: (b, 0, 0),
                         memory_space=pltpu.SMEM),
            pl.BlockSpec((1, 1, _NUM_CLUSTERS), lambda b: (b, 0, 0),
                         memory_space=pltpu.SMEM),
            pl.BlockSpec((1, 3, _NUM_CLUSTERS), lambda b: (b, 0, 0),
                         memory_space=pltpu.SMEM),
            pl.BlockSpec((1, 3, _H, _W), lambda b: (b, 0, 0, 0)),
        ],
        out_specs=pl.BlockSpec((1, _H, _W), lambda b: (b, 0, 0)),
        out_shape=jax.ShapeDtypeStruct((batch, _H, _W), jnp.int32),
        scratch_shapes=[pltpu.VMEM((_H, _W), jnp.float32)],
    )(yc[:, None, :], xc[:, None, :], colors_t, x)


def _seeds(x, grad_map):
    g = grad_map[:, 0]
    ys = np.asarray(_YS, np.int32)
    xs = np.asarray(_XS, np.int32)
    wrows = (ys[:, None] - 10 + np.arange(20)[None, :])
    wcols = (xs[:, None] - 10 + np.arange(20)[None, :])
    rr = np.repeat(wrows, _GRID, axis=0)
    cc = np.tile(wcols, (_GRID, 1))
    win = g[:, rr[:, :, None], cc[:, None, :]]
    flat = win.reshape(win.shape[0], _NUM_CLUSTERS, 400)
    idx = jnp.argmin(flat, axis=-1).astype(jnp.int32)
    ny = jnp.asarray(rr[:, 0])[None, :] + idx // 20
    nx = jnp.asarray(cc[:, 0])[None, :] + idx % 20
    b_idx = jnp.arange(x.shape[0], dtype=jnp.int32)[:, None]
    colors = x[b_idx, :, ny, nx]
    cents = jnp.stack([ny, nx], axis=-1).astype(jnp.float32)
    return cents, ny, nx, colors


def _segment_sums(x, lm):
    batch = x.shape[0]
    rows = jnp.arange(_H, dtype=jnp.float32)[:, None]
    cols = jnp.arange(_W, dtype=jnp.float32)[None, :]
    ones = jnp.ones((batch, _H, _W), jnp.float32)
    data = jnp.stack(
        [ones,
         jnp.broadcast_to(rows, (batch, _H, _W)),
         jnp.broadcast_to(cols, (batch, _H, _W)),
         x[:, 0], x[:, 1], x[:, 2]], axis=-1)
    seg = (lm + 1) + (_NUM_CLUSTERS + 1) * jnp.arange(batch, dtype=jnp.int32)[:, None, None]
    sums = jax.ops.segment_sum(
        data.reshape(-1, 6), seg.reshape(-1),
        num_segments=batch * (_NUM_CLUSTERS + 1))
    sums = sums.reshape(batch, _NUM_CLUSTERS + 1, 6)[:, 1:, :]
    return sums


def _update(yc, xc, colors, sums):
    cnt = sums[..., 0]
    safe = jnp.maximum(cnt, 1.0)
    my = jnp.clip(jnp.rint(sums[..., 1] / safe), 0, _H - 1).astype(jnp.int32)
    mx = jnp.clip(jnp.rint(sums[..., 2] / safe), 0, _W - 1).astype(jnp.int32)
    mc = sums[..., 3:6] / safe[..., None]
    has = cnt > 0
    yc2 = jnp.where(has, my, yc)
    xc2 = jnp.where(has, mx, xc)
    col2 = jnp.where(has[..., None], mc, colors)
    return yc2, xc2, col2


def kernel(x, grad_map):
    if grad_map.ndim == 3:
        grad_map = grad_map[:, None, :, :]
    cents, ny, nx, colors = _seeds(x, grad_map)
    yc, xc = ny, nx
    lm1 = _assign(x, yc, xc, colors)
    sums = _segment_sums(x, lm1)
    yc2, xc2, col2 = _update(yc, xc, colors, sums)
    lm = _assign(x, yc2, xc2, col2)
    return (cents, lm)
